# baseline (speedup 1.0000x reference)
import functools

import jax
import jax.numpy as jnp
from jax import lax
from jax.experimental import pallas as pl
from jax.experimental.pallas import tpu as pltpu
from jax.experimental.pallas import tpu_sc as plsc

SEQ = 4
PACK = 40
PACKR = 48
GW = 128
SW = 2000
MLP_B = 2560


def _gather_call(packed, emb, src2d, dst2d, ea2d):
    n_edges = src2d.shape[1]
    mesh = plsc.VectorSubcoreMesh(core_axis_name="core", subcore_axis_name="subcore")

    @functools.partial(
        pl.kernel,
        out_type=(
            jax.ShapeDtypeStruct((PACKR, n_edges), jnp.float32),
            jax.ShapeDtypeStruct((32, n_edges), jnp.float32),
        ),
        mesh=mesh,
        scratch_types=[
            pltpu.VMEM((GW, PACK), jnp.float32),
            pltpu.VMEM((GW, 32), jnp.float32),
            pltpu.SemaphoreType.DMA,
            pltpu.SemaphoreType.DMA,
        ],
        compiler_params=pltpu.CompilerParams(use_tc_tiling_on_sc=False,
                                             needs_layout_passes=False),
    )
    def gather_kernel(packed_hbm, emb_hbm, src_hbm, dst_hbm, ea_hbm,
                      o1_hbm, o2_hbm, sg, dg, sem1, sem2):
        iota = lax.iota(jnp.int32, 16)

        def body(si, di, eav, o1, o2):
            c1 = pltpu.async_copy(packed_hbm.at[si.at[0]], sg, sem1)
            c2 = pltpu.async_copy(emb_hbm.at[di.at[0]], dg, sem2)
            c1.wait()
            c2.wait()
            for c in range(GW // 16):
                rows = iota + (16 * c)
                sl = pl.ds(16 * c, 16)

                @pl.loop(0, 4)
                def _(j):
                    o1[j, sl] = plsc.load_gather(sg, [rows, jnp.full((16,), 32 + j, jnp.int32)])

                o1[4, sl] = eav[0, sl]
                o1[5, sl] = jnp.zeros((16,), jnp.float32)
                o1[6, sl] = jnp.zeros((16,), jnp.float32)
                o1[7, sl] = jnp.zeros((16,), jnp.float32)

                @pl.loop(8, 40)
                def _(j):
                    o1[j, sl] = plsc.load_gather(sg, [rows, jnp.full((16,), j - 8, jnp.int32)])

                @pl.loop(40, 44)
                def _(j):
                    o1[j, sl] = plsc.load_gather(sg, [rows, jnp.full((16,), j - 4, jnp.int32)])

                o1[44, sl] = jnp.zeros((16,), jnp.float32)
                o1[45, sl] = jnp.zeros((16,), jnp.float32)
                o1[46, sl] = jnp.zeros((16,), jnp.float32)
                o1[47, sl] = jnp.zeros((16,), jnp.float32)

                @pl.loop(0, 32)
                def _(j):
                    o2[j, sl] = plsc.load_gather(dg, [rows, jnp.full((16,), j, jnp.int32)])

        pltpu.emit_pipeline(
            body,
            grid=(n_edges // GW,),
            in_specs=[
                pl.BlockSpec((1, GW), lambda i: (0, i)),
                pl.BlockSpec((1, GW), lambda i: (0, i)),
                pl.BlockSpec((1, GW), lambda i: (0, i)),
            ],
            out_specs=[
                pl.BlockSpec((PACKR, GW), lambda i: (0, i)),
                pl.BlockSpec((32, GW), lambda i: (0, i)),
            ],
            core_axis_name=("core", "subcore"),
            dimension_semantics=(pltpu.PARALLEL,),
        )(src_hbm, dst_hbm, ea_hbm, o1_hbm, o2_hbm)

    return gather_kernel(packed, emb, src2d, dst2d, ea2d)


def _mlp_body(f1_ref, f2_ref, w4a_ref, w4b_ref, r4_ref, w2b_ref,
              p_ref, q_ref):
    f1 = f1_ref[...]
    prod = f1[8:40, :] * f2_ref[...]
    h4 = (lax.dot_general(w4a_ref[...], f1[0:8, :], (((1,), (0,)), ((), ())),
                          preferred_element_type=jnp.float32)
          + lax.dot_general(w4b_ref[...], prod, (((1,), (0,)), ((), ())),
                            preferred_element_type=jnp.float32)
          + r4_ref[...])
    h4 = jnp.maximum(h4, 0.0)
    logits = lax.dot_general(w2b_ref[...], h4, (((1,), (0,)), ((), ())),
                             preferred_element_type=jnp.float32)
    ex = jnp.exp(logits)
    p_ref[...] = jnp.sum(ex, axis=0, keepdims=True)
    q_ref[...] = jnp.sum(ex * f1[40:44, :], axis=0, keepdims=True)


def _mlp_call(f1, f2, w4a, w4b, r4, w2b):
    n_edges = f1.shape[1]
    grid = (n_edges // MLP_B,)
    full = lambda shape: pl.BlockSpec(shape, lambda i: tuple(0 for _ in shape))
    return pl.pallas_call(
        _mlp_body,
        grid=grid,
        in_specs=[
            pl.BlockSpec((PACKR, MLP_B), lambda i: (0, i)),
            pl.BlockSpec((32, MLP_B), lambda i: (0, i)),
            full((4 * 64, 8)),
            full((4 * 64, 32)),
            full((4 * 64, 1)),
            full((SEQ, 4 * 64)),
        ],
        out_specs=[
            pl.BlockSpec((1, MLP_B), lambda i: (0, i)),
            pl.BlockSpec((1, MLP_B), lambda i: (0, i)),
        ],
        out_shape=(
            jax.ShapeDtypeStruct((1, n_edges), jnp.float32),
            jax.ShapeDtypeStruct((1, n_edges), jnp.float32),
        ),
    )(f1, f2, w4a, w4b, r4, w2b)


def _scatter_call(dst_g, p_g, q_g, n_nodes):
    n_win, _ = dst_g.shape
    n_workers = 32
    acc_len = 2 * n_nodes
    mesh = plsc.VectorSubcoreMesh(core_axis_name="core", subcore_axis_name="subcore")

    @functools.partial(
        pl.kernel,
        out_type=jax.ShapeDtypeStruct((n_workers, acc_len), jnp.float32),
        mesh=mesh,
        scratch_types=[pltpu.VMEM((acc_len,), jnp.float32)],
        compiler_params=pltpu.CompilerParams(needs_layout_passes=False),
    )
    def scatter_kernel(dst_hbm, p_hbm, q_hbm, out_hbm, acc):
        wid = lax.axis_index("subcore") * 2 + lax.axis_index("core")

        @pl.loop(0, acc_len, step=16)
        def _(i):
            acc[pl.ds(i, 16)] = jnp.zeros((16,), jnp.float32)

        def body(dv, pv, qv):
            @pl.loop(0, SW, step=16)
            def _(j):
                idx = dv[0, pl.ds(j, 16)]
                plsc.addupdate_scatter(acc, [idx], pv[0, pl.ds(j, 16)])
                plsc.addupdate_scatter(acc, [idx + n_nodes], qv[0, pl.ds(j, 16)])

        pltpu.emit_pipeline(
            body,
            grid=(n_win,),
            in_specs=[
                pl.BlockSpec((1, SW), lambda i: (i, 0)),
                pl.BlockSpec((1, SW), lambda i: (i, 0)),
                pl.BlockSpec((1, SW), lambda i: (i, 0)),
            ],
            core_axis_name=("core", "subcore"),
            dimension_semantics=(pltpu.PARALLEL,),
        )(dst_hbm, p_hbm, q_hbm)

        pltpu.sync_copy(acc, out_hbm.at[wid])

    return scatter_kernel(dst_g, p_g, q_g)


def _combine_body(parts_ref, out_ref):
    s = jnp.sum(parts_ref[...], axis=0)
    out_ref[...] = s[1] / (s[0] + 1e-16)


def _combine_call(parts3, n_nodes):
    n_workers = parts3.shape[0]
    return pl.pallas_call(
        _combine_body,
        in_specs=[pl.BlockSpec((n_workers, 2, n_nodes), lambda: (0, 0, 0))],
        out_specs=pl.BlockSpec((n_nodes,), lambda: (0,)),
        out_shape=jax.ShapeDtypeStruct((n_nodes,), jnp.float32),
    )(parts3)


def kernel(y, hist_average_y, avg_speed, time_feats, edge_attr, emb,
           W1, b1, W2, b2, Wd1, bd1, Wd2, bd2, edge_index):
    n_nodes = emb.shape[0]
    n_edges = edge_index.shape[1]

    x_t = y[:SEQ].T
    av_t = avg_speed[:SEQ].T
    packed = jnp.concatenate([emb, av_t, x_t], axis=1)

    src2d = edge_index[0].reshape(1, n_edges)
    dst2d = edge_index[1].reshape(1, n_edges)

    w4a = jnp.concatenate(
        [jnp.kron(jnp.eye(SEQ, dtype=jnp.float32), W1[0].reshape(64, 1)),
         jnp.tile(W1[5].reshape(64, 1), (SEQ, 1)),
         jnp.zeros((SEQ * 64, 3), jnp.float32)], axis=1)
    w4b = jnp.tile(W1[6:38].T, (SEQ, 1))
    r4 = (W1[1:5] + b1[None, :]).reshape(SEQ * 64, 1)
    w2b = jnp.kron(jnp.eye(SEQ, dtype=jnp.float32), W2[:, 0].reshape(1, 64))

    ea2d = edge_attr.reshape(1, n_edges)
    f1, f2 = _gather_call(packed, emb, src2d, dst2d, ea2d)
    p, q = _mlp_call(f1, f2, w4a, w4b, r4, w2b)

    dst_g = edge_index[1].reshape(n_edges // SW, SW)
    p_g = p.reshape(n_edges // SW, SW)
    q_g = q.reshape(n_edges // SW, SW)
    parts = _scatter_call(dst_g, p_g, q_g, n_nodes)

    return _combine_call(parts.reshape(32, 2, n_nodes), n_nodes)


# speedup vs baseline: 1.0870x; 1.0870x over previous
import functools

import jax
import jax.numpy as jnp
from jax import lax
from jax.experimental import pallas as pl
from jax.experimental.pallas import tpu as pltpu
from jax.experimental.pallas import tpu_sc as plsc

SEQ = 4
PACK = 40
PACKR = 48
GW = 128
SW = 2000
MLP_B = 2560


def _gather_call(packed, emb, src2d, dst2d, ea2d):
    n_edges = src2d.shape[1]
    mesh = plsc.VectorSubcoreMesh(core_axis_name="core", subcore_axis_name="subcore")

    @functools.partial(
        pl.kernel,
        out_type=jax.ShapeDtypeStruct((PACKR, n_edges), jnp.float32),
        mesh=mesh,
        scratch_types=[
            pltpu.VMEM((GW, PACK), jnp.float32),
            pltpu.VMEM((GW, 32), jnp.float32),
            pltpu.SemaphoreType.DMA,
            pltpu.SemaphoreType.DMA,
        ],
        compiler_params=pltpu.CompilerParams(use_tc_tiling_on_sc=False,
                                             needs_layout_passes=False),
    )
    def gather_kernel(packed_hbm, emb_hbm, src_hbm, dst_hbm, ea_hbm,
                      o1_hbm, sg, dg, sem1, sem2):
        iota = lax.iota(jnp.int32, 16)
        zeros16 = jnp.zeros((16,), jnp.float32)

        def body(si, di, eav, o1):
            c1 = pltpu.async_copy(packed_hbm.at[si.at[0]], sg, sem1)
            c2 = pltpu.async_copy(emb_hbm.at[di.at[0]], dg, sem2)
            c1.wait()
            c2.wait()
            for c in range(GW // 16):
                rows = iota + (16 * c)
                sl = pl.ds(16 * c, 16)
                for j in range(4):
                    col = jnp.full((16,), 32 + j, jnp.int32)
                    o1[j, sl] = plsc.load_gather(sg, [rows, col])
                o1[4, sl] = eav[0, sl]
                for j in range(5, 8):
                    o1[j, sl] = zeros16
                for j in range(32):
                    col = jnp.full((16,), j, jnp.int32)
                    o1[8 + j, sl] = (plsc.load_gather(sg, [rows, col]) *
                                     plsc.load_gather(dg, [rows, col]))
                for j in range(4):
                    col = jnp.full((16,), 36 + j, jnp.int32)
                    o1[40 + j, sl] = plsc.load_gather(sg, [rows, col])
                for j in range(44, 48):
                    o1[j, sl] = zeros16

        pltpu.emit_pipeline(
            body,
            grid=(n_edges // GW,),
            in_specs=[
                pl.BlockSpec((1, GW), lambda i: (0, i)),
                pl.BlockSpec((1, GW), lambda i: (0, i)),
                pl.BlockSpec((1, GW), lambda i: (0, i)),
            ],
            out_specs=[
                pl.BlockSpec((PACKR, GW), lambda i: (0, i)),
            ],
            core_axis_name=("core", "subcore"),
            dimension_semantics=(pltpu.PARALLEL,),
        )(src_hbm, dst_hbm, ea_hbm, o1_hbm)

    return gather_kernel(packed, emb, src2d, dst2d, ea2d)


def _mlp_body(f1_ref, w4a_ref, w4b_ref, r4_ref, w2b_ref, p_ref, q_ref):
    f1 = f1_ref[...]
    h4 = (lax.dot_general(w4a_ref[...], f1[0:8, :], (((1,), (0,)), ((), ())),
                          preferred_element_type=jnp.float32)
          + lax.dot_general(w4b_ref[...], f1[8:40, :], (((1,), (0,)), ((), ())),
                            preferred_element_type=jnp.float32)
          + r4_ref[...])
    h4 = jnp.maximum(h4, 0.0)
    logits = lax.dot_general(w2b_ref[...], h4, (((1,), (0,)), ((), ())),
                             preferred_element_type=jnp.float32)
    ex = jnp.exp(logits)
    p_ref[...] = jnp.sum(ex, axis=0, keepdims=True)
    q_ref[...] = jnp.sum(ex * f1[40:44, :], axis=0, keepdims=True)


def _mlp_call(f1, w4a, w4b, r4, w2b):
    n_edges = f1.shape[1]
    grid = (n_edges // MLP_B,)
    full = lambda shape: pl.BlockSpec(shape, lambda i: tuple(0 for _ in shape))
    return pl.pallas_call(
        _mlp_body,
        grid=grid,
        in_specs=[
            pl.BlockSpec((PACKR, MLP_B), lambda i: (0, i)),
            full((4 * 64, 8)),
            full((4 * 64, 32)),
            full((4 * 64, 1)),
            full((SEQ, 4 * 64)),
        ],
        out_specs=[
            pl.BlockSpec((1, MLP_B), lambda i: (0, i)),
            pl.BlockSpec((1, MLP_B), lambda i: (0, i)),
        ],
        out_shape=(
            jax.ShapeDtypeStruct((1, n_edges), jnp.float32),
            jax.ShapeDtypeStruct((1, n_edges), jnp.float32),
        ),
    )(f1, w4a, w4b, r4, w2b)


def _scatter_call(dst_g, p_g, q_g, n_nodes):
    n_win, _ = dst_g.shape
    n_workers = 32
    acc_len = 2 * n_nodes
    mesh = plsc.VectorSubcoreMesh(core_axis_name="core", subcore_axis_name="subcore")

    @functools.partial(
        pl.kernel,
        out_type=jax.ShapeDtypeStruct((n_workers, acc_len), jnp.float32),
        mesh=mesh,
        scratch_types=[pltpu.VMEM((acc_len,), jnp.float32)],
        compiler_params=pltpu.CompilerParams(needs_layout_passes=False),
    )
    def scatter_kernel(dst_hbm, p_hbm, q_hbm, out_hbm, acc):
        wid = lax.axis_index("subcore") * 2 + lax.axis_index("core")

        @pl.loop(0, acc_len, step=16)
        def _(i):
            acc[pl.ds(i, 16)] = jnp.zeros((16,), jnp.float32)

        def body(dv, pv, qv):
            @pl.loop(0, SW, step=16)
            def _(j):
                idx = dv[0, pl.ds(j, 16)]
                plsc.addupdate_scatter(acc, [idx], pv[0, pl.ds(j, 16)])
                plsc.addupdate_scatter(acc, [idx + n_nodes], qv[0, pl.ds(j, 16)])

        pltpu.emit_pipeline(
            body,
            grid=(n_win,),
            in_specs=[
                pl.BlockSpec((1, SW), lambda i: (i, 0)),
                pl.BlockSpec((1, SW), lambda i: (i, 0)),
                pl.BlockSpec((1, SW), lambda i: (i, 0)),
            ],
            core_axis_name=("core", "subcore"),
            dimension_semantics=(pltpu.PARALLEL,),
        )(dst_hbm, p_hbm, q_hbm)

        pltpu.sync_copy(acc, out_hbm.at[wid])

    return scatter_kernel(dst_g, p_g, q_g)


def _combine_body(parts_ref, out_ref):
    s = jnp.sum(parts_ref[...], axis=0)
    out_ref[...] = s[1] / (s[0] + 1e-16)


def _combine_call(parts3, n_nodes):
    n_workers = parts3.shape[0]
    return pl.pallas_call(
        _combine_body,
        in_specs=[pl.BlockSpec((n_workers, 2, n_nodes), lambda: (0, 0, 0))],
        out_specs=pl.BlockSpec((n_nodes,), lambda: (0,)),
        out_shape=jax.ShapeDtypeStruct((n_nodes,), jnp.float32),
    )(parts3)


def kernel(y, hist_average_y, avg_speed, time_feats, edge_attr, emb,
           W1, b1, W2, b2, Wd1, bd1, Wd2, bd2, edge_index):
    n_nodes = emb.shape[0]
    n_edges = edge_index.shape[1]

    x_t = y[:SEQ].T
    av_t = avg_speed[:SEQ].T
    packed = jnp.concatenate([emb, av_t, x_t], axis=1)

    src2d = edge_index[0].reshape(1, n_edges)
    dst2d = edge_index[1].reshape(1, n_edges)

    w4a = jnp.concatenate(
        [jnp.kron(jnp.eye(SEQ, dtype=jnp.float32), W1[0].reshape(64, 1)),
         jnp.tile(W1[5].reshape(64, 1), (SEQ, 1)),
         jnp.zeros((SEQ * 64, 3), jnp.float32)], axis=1)
    w4b = jnp.tile(W1[6:38].T, (SEQ, 1))
    r4 = (W1[1:5] + b1[None, :]).reshape(SEQ * 64, 1)
    w2b = jnp.kron(jnp.eye(SEQ, dtype=jnp.float32), W2[:, 0].reshape(1, 64))

    ea2d = edge_attr.reshape(1, n_edges)
    f1 = _gather_call(packed, emb, src2d, dst2d, ea2d)
    p, q = _mlp_call(f1, w4a, w4b, r4, w2b)

    dst_g = edge_index[1].reshape(n_edges // SW, SW)
    p_g = p.reshape(n_edges // SW, SW)
    q_g = q.reshape(n_edges // SW, SW)
    parts = _scatter_call(dst_g, p_g, q_g, n_nodes)

    return _combine_call(parts.reshape(32, 2, n_nodes), n_nodes)


# speedup vs baseline: 1.2105x; 1.1136x over previous
import functools

import jax
import jax.numpy as jnp
from jax import lax
from jax.experimental import pallas as pl
from jax.experimental.pallas import tpu as pltpu
from jax.experimental.pallas import tpu_sc as plsc

SEQ = 4
PACK = 40
PACKR = 48
GW = 128
SW = 2000
MLP_B = 3200
N_CHUNK = 4


def _gather_call(packed, emb, src2d, dst2d, ea2d):
    n_edges = src2d.shape[1]
    mesh = plsc.VectorSubcoreMesh(core_axis_name="core", subcore_axis_name="subcore")

    @functools.partial(
        pl.kernel,
        out_type=jax.ShapeDtypeStruct((PACKR, n_edges), jnp.float32),
        mesh=mesh,
        scratch_types=[
            pltpu.VMEM((GW, PACK), jnp.float32),
            pltpu.VMEM((GW, 32), jnp.float32),
            pltpu.SemaphoreType.DMA,
            pltpu.SemaphoreType.DMA,
        ],
        compiler_params=pltpu.CompilerParams(use_tc_tiling_on_sc=False,
                                             needs_layout_passes=False),
    )
    def gather_kernel(packed_hbm, emb_hbm, src_hbm, dst_hbm, ea_hbm,
                      o1_hbm, sg, dg, sem1, sem2):
        iota = lax.iota(jnp.int32, 16)
        zeros16 = jnp.zeros((16,), jnp.float32)

        def body(si, di, eav, o1):
            c1 = pltpu.async_copy(packed_hbm.at[si.at[0]], sg, sem1)
            c2 = pltpu.async_copy(emb_hbm.at[di.at[0]], dg, sem2)
            c1.wait()
            c2.wait()
            for c in range(GW // 16):
                rows = iota + (16 * c)
                sl = pl.ds(16 * c, 16)
                for j in range(4):
                    col = jnp.full((16,), 32 + j, jnp.int32)
                    o1[j, sl] = plsc.load_gather(sg, [rows, col])
                o1[4, sl] = eav[0, sl]
                for j in range(5, 8):
                    o1[j, sl] = zeros16
                for j in range(32):
                    col = jnp.full((16,), j, jnp.int32)
                    o1[8 + j, sl] = (plsc.load_gather(sg, [rows, col]) *
                                     plsc.load_gather(dg, [rows, col]))
                for j in range(4):
                    col = jnp.full((16,), 36 + j, jnp.int32)
                    o1[40 + j, sl] = plsc.load_gather(sg, [rows, col])
                for j in range(44, 48):
                    o1[j, sl] = zeros16

        pltpu.emit_pipeline(
            body,
            grid=(n_edges // GW,),
            in_specs=[
                pl.BlockSpec((1, GW), lambda i: (0, i)),
                pl.BlockSpec((1, GW), lambda i: (0, i)),
                pl.BlockSpec((1, GW), lambda i: (0, i)),
            ],
            out_specs=[
                pl.BlockSpec((PACKR, GW), lambda i: (0, i)),
            ],
            core_axis_name=("core", "subcore"),
            dimension_semantics=(pltpu.PARALLEL,),
        )(src_hbm, dst_hbm, ea_hbm, o1_hbm)

    return gather_kernel(packed, emb, src2d, dst2d, ea2d)


def _mlp_body(f1_ref, w4a_ref, w4b_ref, r4_ref, w2b_ref, p_ref, q_ref):
    f1 = f1_ref[...]
    h4 = (lax.dot_general(w4a_ref[...], f1[0:8, :], (((1,), (0,)), ((), ())),
                          preferred_element_type=jnp.float32)
          + lax.dot_general(w4b_ref[...], f1[8:40, :], (((1,), (0,)), ((), ())),
                            preferred_element_type=jnp.float32)
          + r4_ref[...])
    h4 = jnp.maximum(h4, 0.0)
    logits = lax.dot_general(w2b_ref[...], h4, (((1,), (0,)), ((), ())),
                             preferred_element_type=jnp.float32)
    ex = jnp.exp(logits)
    p_ref[...] = jnp.sum(ex, axis=0, keepdims=True)
    q_ref[...] = jnp.sum(ex * f1[40:44, :], axis=0, keepdims=True)


def _mlp_call(f1, w4a, w4b, r4, w2b):
    n_edges = f1.shape[1]
    grid = (n_edges // MLP_B,)
    full = lambda shape: pl.BlockSpec(shape, lambda i: tuple(0 for _ in shape))
    return pl.pallas_call(
        _mlp_body,
        grid=grid,
        in_specs=[
            pl.BlockSpec((PACKR, MLP_B), lambda i: (0, i)),
            full((4 * 64, 8)),
            full((4 * 64, 32)),
            full((4 * 64, 1)),
            full((SEQ, 4 * 64)),
        ],
        out_specs=[
            pl.BlockSpec((1, MLP_B), lambda i: (0, i)),
            pl.BlockSpec((1, MLP_B), lambda i: (0, i)),
        ],
        out_shape=(
            jax.ShapeDtypeStruct((1, n_edges), jnp.float32),
            jax.ShapeDtypeStruct((1, n_edges), jnp.float32),
        ),
    )(f1, w4a, w4b, r4, w2b)


def _scatter_call(dst_g, p_g, q_g, n_nodes):
    n_win, _ = dst_g.shape
    n_workers = 32
    acc_len = 2 * n_nodes
    mesh = plsc.VectorSubcoreMesh(core_axis_name="core", subcore_axis_name="subcore")

    @functools.partial(
        pl.kernel,
        out_type=jax.ShapeDtypeStruct((n_workers, acc_len), jnp.float32),
        mesh=mesh,
        scratch_types=[pltpu.VMEM((acc_len,), jnp.float32)],
        compiler_params=pltpu.CompilerParams(needs_layout_passes=False),
    )
    def scatter_kernel(dst_hbm, p_hbm, q_hbm, out_hbm, acc):
        wid = lax.axis_index("subcore") * 2 + lax.axis_index("core")

        @pl.loop(0, acc_len, step=16)
        def _(i):
            acc[pl.ds(i, 16)] = jnp.zeros((16,), jnp.float32)

        def body(dv, pv, qv):
            @pl.loop(0, SW, step=16)
            def _(j):
                idx = dv[0, pl.ds(j, 16)]
                plsc.addupdate_scatter(acc, [idx], pv[0, pl.ds(j, 16)])
                plsc.addupdate_scatter(acc, [idx + n_nodes], qv[0, pl.ds(j, 16)])

        pltpu.emit_pipeline(
            body,
            grid=(n_win,),
            in_specs=[
                pl.BlockSpec((1, SW), lambda i: (i, 0)),
                pl.BlockSpec((1, SW), lambda i: (i, 0)),
                pl.BlockSpec((1, SW), lambda i: (i, 0)),
            ],
            core_axis_name=("core", "subcore"),
            dimension_semantics=(pltpu.PARALLEL,),
        )(dst_hbm, p_hbm, q_hbm)

        pltpu.sync_copy(acc, out_hbm.at[wid])

    return scatter_kernel(dst_g, p_g, q_g)


def _combine_body(parts_ref, out_ref):
    s = jnp.sum(parts_ref[...], axis=0)
    out_ref[...] = s[1] / (s[0] + 1e-16)


def _combine_call(parts3, n_nodes):
    n_workers = parts3.shape[0]
    return pl.pallas_call(
        _combine_body,
        in_specs=[pl.BlockSpec((n_workers, 2, n_nodes), lambda: (0, 0, 0))],
        out_specs=pl.BlockSpec((n_nodes,), lambda: (0,)),
        out_shape=jax.ShapeDtypeStruct((n_nodes,), jnp.float32),
    )(parts3)


def kernel(y, hist_average_y, avg_speed, time_feats, edge_attr, emb,
           W1, b1, W2, b2, Wd1, bd1, Wd2, bd2, edge_index):
    n_nodes = emb.shape[0]
    n_edges = edge_index.shape[1]

    x_t = y[:SEQ].T
    av_t = avg_speed[:SEQ].T
    packed = jnp.concatenate([emb, av_t, x_t], axis=1)

    src2d = edge_index[0].reshape(1, n_edges)
    dst2d = edge_index[1].reshape(1, n_edges)

    w4a = jnp.concatenate(
        [jnp.kron(jnp.eye(SEQ, dtype=jnp.float32), W1[0].reshape(64, 1)),
         jnp.tile(W1[5].reshape(64, 1), (SEQ, 1)),
         jnp.zeros((SEQ * 64, 3), jnp.float32)], axis=1)
    w4b = jnp.tile(W1[6:38].T, (SEQ, 1))
    r4 = (W1[1:5] + b1[None, :]).reshape(SEQ * 64, 1)
    w2b = jnp.kron(jnp.eye(SEQ, dtype=jnp.float32), W2[:, 0].reshape(1, 64))

    ec = n_edges // N_CHUNK
    ps, qs = [], []
    for k in range(N_CHUNK):
        sl = slice(k * ec, (k + 1) * ec)
        f1 = _gather_call(packed, emb,
                          edge_index[0, sl].reshape(1, ec),
                          edge_index[1, sl].reshape(1, ec),
                          edge_attr[sl].reshape(1, ec))
        p, q = _mlp_call(f1, w4a, w4b, r4, w2b)
        ps.append(p)
        qs.append(q)
    p = jnp.concatenate(ps, axis=1)
    q = jnp.concatenate(qs, axis=1)

    dst_g = edge_index[1].reshape(n_edges // SW, SW)
    p_g = p.reshape(n_edges // SW, SW)
    q_g = q.reshape(n_edges // SW, SW)
    parts = _scatter_call(dst_g, p_g, q_g, n_nodes)

    return _combine_call(parts.reshape(32, 2, n_nodes), n_nodes)
